# baseline (speedup 1.0000x reference)
import functools

import jax
import jax.numpy as jnp
from jax import lax
from jax.experimental import pallas as pl
from jax.experimental.pallas import tpu as pltpu
from jax.experimental.pallas import tpu_sc as plsc

N_EXPERTS = 64
TOPK = 8
BT = 1024
NW = 32
L = 16


def _tc_scores_body(x_ref, wt_ref, st_ref):
    s = jnp.dot(x_ref[...], wt_ref[...], preferred_element_type=jnp.float32)
    st_ref[...] = s.T


def _tc_scores_t(x, wt):
    n_tokens, dim = x.shape
    return pl.pallas_call(
        _tc_scores_body,
        grid=(n_tokens // BT,),
        in_specs=[
            pl.BlockSpec((BT, dim), lambda i: (i, 0)),
            pl.BlockSpec((dim, N_EXPERTS), lambda i: (0, 0)),
        ],
        out_specs=pl.BlockSpec((N_EXPERTS, BT), lambda i: (0, i)),
        out_shape=jax.ShapeDtypeStruct((N_EXPERTS, n_tokens), jnp.float32),
    )(x, wt)


def _sc_topk(scores_t):
    n_tokens = scores_t.shape[1]
    nt = n_tokens // NW

    @functools.partial(
        pl.kernel,
        mesh=plsc.VectorSubcoreMesh(core_axis_name="c", subcore_axis_name="s"),
        out_type=[
            jax.ShapeDtypeStruct((TOPK, n_tokens), jnp.int32),
            jax.ShapeDtypeStruct((TOPK, n_tokens), jnp.float32),
        ],
        scratch_types=[
            pltpu.VMEM((N_EXPERTS, nt), jnp.float32),
            pltpu.VMEM((TOPK, nt), jnp.int32),
            pltpu.VMEM((TOPK, nt), jnp.float32),
        ],
    )
    def topk_kernel(s_hbm, topi_hbm, topv_hbm, sbuf, ibuf, vbuf):
        wid = lax.axis_index("s") * 2 + lax.axis_index("c")
        base = wid * nt
        pltpu.sync_copy(s_hbm.at[:, pl.ds(base, nt)], sbuf)

        GPI = 1

        def group(g, _):
            los = [(g * GPI + q) * L for q in range(GPI)]
            dsum = [jnp.zeros((L,), jnp.float32) for _ in range(GPI)]
            t = [[jnp.full((L,), -1.0, jnp.float32) for _ in range(TOPK)]
                 for _ in range(GPI)]
            for e in range(N_EXPERTS):
                for q in range(GPI):
                    sv = sbuf[e, pl.ds(los[q], L)]
                    ev = jnp.exp(sv)
                    dsum[q] = dsum[q] + ev
                    kb = ((lax.bitcast_convert_type(ev, jnp.int32)
                           & -N_EXPERTS) | (N_EXPERTS - 1 - e))
                    c = lax.bitcast_convert_type(kb, jnp.float32)
                    for j in range(TOPK):
                        hi = jnp.maximum(t[q][j], c)
                        c = jnp.minimum(t[q][j], c)
                        t[q][j] = hi
            for q in range(GPI):
                rinv = 1.0 / dsum[q]
                for j in range(TOPK):
                    tb = lax.bitcast_convert_type(t[q][j], jnp.int32)
                    ibuf[j, pl.ds(los[q], L)] = (
                        (N_EXPERTS - 1) - (tb & (N_EXPERTS - 1)))
                    vbuf[j, pl.ds(los[q], L)] = (
                        lax.bitcast_convert_type(tb & -N_EXPERTS, jnp.float32)
                        * rinv)
            return 0

        lax.fori_loop(0, nt // (L * GPI), group, 0)
        pltpu.sync_copy(ibuf, topi_hbm.at[:, pl.ds(base, nt)])
        pltpu.sync_copy(vbuf, topv_hbm.at[:, pl.ds(base, nt)])

    return topk_kernel(scores_t)


NCHUNK = 2


@jax.jit
def kernel(x, W):
    n_tokens = x.shape[0]
    wt = W.T
    ct = n_tokens // NCHUNK
    tis, tvs = [], []
    for c in range(NCHUNK):
        scores_t = _tc_scores_t(x[c * ct:(c + 1) * ct], wt)
        topi_t, topv_t = _sc_topk(scores_t)
        tis.append(topi_t.T)
        tvs.append(topv_t.T)
    return (jnp.concatenate(tis, axis=0), jnp.concatenate(tvs, axis=0))


# speedup vs baseline: 2.5992x; 2.5992x over previous
import functools

import jax
import jax.numpy as jnp
from jax import lax
from jax.experimental import pallas as pl
from jax.experimental.pallas import tpu as pltpu
from jax.experimental.pallas import tpu_sc as plsc

N_EXPERTS = 64
TOPK = 8
BT = 1024
NW = 32
L = 16


def _tc_scores_body(x_ref, wt_ref, st_ref):
    s = jnp.dot(x_ref[...], wt_ref[...], preferred_element_type=jnp.float32)
    st_ref[...] = s.T


def _tc_scores_t(x, wt, chunk_tokens, chunk_idx):
    dim = x.shape[1]
    blk0 = chunk_idx * (chunk_tokens // BT)
    return pl.pallas_call(
        _tc_scores_body,
        grid=(chunk_tokens // BT,),
        in_specs=[
            pl.BlockSpec((BT, dim), lambda i: (blk0 + i, 0)),
            pl.BlockSpec((dim, N_EXPERTS), lambda i: (0, 0)),
        ],
        out_specs=pl.BlockSpec((N_EXPERTS, BT), lambda i: (0, i)),
        out_shape=jax.ShapeDtypeStruct((N_EXPERTS, chunk_tokens), jnp.float32),
    )(x, wt)


def _sc_topk(scores_t):
    n_tokens = scores_t.shape[1]
    nt = n_tokens // NW

    @functools.partial(
        pl.kernel,
        mesh=plsc.VectorSubcoreMesh(core_axis_name="c", subcore_axis_name="s"),
        out_type=[
            jax.ShapeDtypeStruct((TOPK, n_tokens), jnp.int32),
            jax.ShapeDtypeStruct((TOPK, n_tokens), jnp.float32),
        ],
        scratch_types=[
            pltpu.VMEM((N_EXPERTS, nt), jnp.float32),
            pltpu.VMEM((TOPK, nt), jnp.int32),
            pltpu.VMEM((TOPK, nt), jnp.float32),
        ],
    )
    def topk_kernel(s_hbm, topi_hbm, topv_hbm, sbuf, ibuf, vbuf):
        wid = lax.axis_index("s") * 2 + lax.axis_index("c")
        base = wid * nt
        pltpu.sync_copy(s_hbm.at[:, pl.ds(base, nt)], sbuf)

        GPI = 1

        def group(g, _):
            los = [(g * GPI + q) * L for q in range(GPI)]
            dsum = [jnp.zeros((L,), jnp.float32) for _ in range(GPI)]
            t = [[jnp.full((L,), -1.0, jnp.float32) for _ in range(TOPK)]
                 for _ in range(GPI)]
            for e in range(N_EXPERTS):
                for q in range(GPI):
                    sv = sbuf[e, pl.ds(los[q], L)]
                    ev = jnp.exp(sv)
                    dsum[q] = dsum[q] + ev
                    kb = ((lax.bitcast_convert_type(ev, jnp.int32)
                           & -N_EXPERTS) | (N_EXPERTS - 1 - e))
                    c = lax.bitcast_convert_type(kb, jnp.float32)
                    for j in range(TOPK):
                        hi = jnp.maximum(t[q][j], c)
                        c = jnp.minimum(t[q][j], c)
                        t[q][j] = hi
            for q in range(GPI):
                rinv = 1.0 / dsum[q]
                for j in range(TOPK):
                    tb = lax.bitcast_convert_type(t[q][j], jnp.int32)
                    ibuf[j, pl.ds(los[q], L)] = (
                        (N_EXPERTS - 1) - (tb & (N_EXPERTS - 1)))
                    vbuf[j, pl.ds(los[q], L)] = (
                        lax.bitcast_convert_type(tb & -N_EXPERTS, jnp.float32)
                        * rinv)
            return 0

        lax.fori_loop(0, nt // (L * GPI), group, 0)
        pltpu.sync_copy(ibuf, topi_hbm.at[:, pl.ds(base, nt)])
        pltpu.sync_copy(vbuf, topv_hbm.at[:, pl.ds(base, nt)])

    return topk_kernel(scores_t)


NCHUNK = 2


@jax.jit
def kernel(x, W):
    n_tokens = x.shape[0]
    wt = W.T
    ct = n_tokens // NCHUNK
    tis, tvs = [], []
    for c in range(NCHUNK):
        scores_t = _tc_scores_t(x, wt, ct, c)
        topi_t, topv_t = _sc_topk(scores_t)
        tis.append(topi_t.T)
        tvs.append(topv_t.T)
    return (jnp.concatenate(tis, axis=0), jnp.concatenate(tvs, axis=0))


# speedup vs baseline: 2.6271x; 1.0107x over previous
import functools

import jax
import jax.numpy as jnp
from jax import lax
from jax.experimental import pallas as pl
from jax.experimental.pallas import tpu as pltpu
from jax.experimental.pallas import tpu_sc as plsc

N_EXPERTS = 64
TOPK = 8
BT = 1024
NW = 32
L = 16


def _tc_scores_body(x_ref, wt_ref, st_ref):
    s = jnp.dot(x_ref[...], wt_ref[...], preferred_element_type=jnp.float32)
    st_ref[...] = s.T


def _tc_scores_t(x, wt, chunk_tokens, chunk_idx):
    dim = x.shape[1]
    blk0 = chunk_idx * (chunk_tokens // BT)
    return pl.pallas_call(
        _tc_scores_body,
        grid=(chunk_tokens // BT,),
        in_specs=[
            pl.BlockSpec((BT, dim), lambda i: (blk0 + i, 0)),
            pl.BlockSpec((dim, N_EXPERTS), lambda i: (0, 0)),
        ],
        out_specs=pl.BlockSpec((N_EXPERTS, BT), lambda i: (0, i)),
        out_shape=jax.ShapeDtypeStruct((N_EXPERTS, chunk_tokens), jnp.float32),
    )(x, wt)


def _sc_topk(scores_t):
    n_tokens = scores_t.shape[1]
    nt = n_tokens // NW

    @functools.partial(
        pl.kernel,
        mesh=plsc.VectorSubcoreMesh(core_axis_name="c", subcore_axis_name="s"),
        out_type=[
            jax.ShapeDtypeStruct((TOPK, n_tokens), jnp.int32),
            jax.ShapeDtypeStruct((TOPK, n_tokens), jnp.float32),
        ],
        scratch_types=[
            pltpu.VMEM((N_EXPERTS, nt), jnp.float32),
            pltpu.VMEM((TOPK, nt), jnp.int32),
            pltpu.VMEM((TOPK, nt), jnp.float32),
        ],
    )
    def topk_kernel(s_hbm, topi_hbm, topv_hbm, sbuf, ibuf, vbuf):
        wid = lax.axis_index("s") * 2 + lax.axis_index("c")
        base = wid * nt
        pltpu.sync_copy(s_hbm.at[:, pl.ds(base, nt)], sbuf)

        GPI = 1

        def group(g, _):
            los = [(g * GPI + q) * L for q in range(GPI)]
            dsum = [jnp.zeros((L,), jnp.float32) for _ in range(GPI)]
            t = [[jnp.full((L,), -1.0, jnp.float32) for _ in range(TOPK)]
                 for _ in range(GPI)]
            for e in range(N_EXPERTS):
                for q in range(GPI):
                    sv = sbuf[e, pl.ds(los[q], L)]
                    ev = jnp.exp(sv)
                    dsum[q] = dsum[q] + ev
                    kb = ((lax.bitcast_convert_type(ev, jnp.int32)
                           & -N_EXPERTS) | (N_EXPERTS - 1 - e))
                    c = lax.bitcast_convert_type(kb, jnp.float32)
                    for j in range(TOPK):
                        hi = jnp.maximum(t[q][j], c)
                        c = jnp.minimum(t[q][j], c)
                        t[q][j] = hi
            for q in range(GPI):
                rinv = 1.0 / dsum[q]
                for j in range(TOPK):
                    tb = lax.bitcast_convert_type(t[q][j], jnp.int32)
                    ibuf[j, pl.ds(los[q], L)] = (
                        (N_EXPERTS - 1) - (tb & (N_EXPERTS - 1)))
                    vbuf[j, pl.ds(los[q], L)] = (
                        lax.bitcast_convert_type(tb & -N_EXPERTS, jnp.float32)
                        * rinv)
            return 0

        lax.fori_loop(0, nt // (L * GPI), group, 0)
        pltpu.sync_copy(ibuf, topi_hbm.at[:, pl.ds(base, nt)])
        pltpu.sync_copy(vbuf, topv_hbm.at[:, pl.ds(base, nt)])

    return topk_kernel(scores_t)


NCHUNK = 1


@jax.jit
def kernel(x, W):
    n_tokens = x.shape[0]
    wt = W.T
    ct = n_tokens // NCHUNK
    tis, tvs = [], []
    for c in range(NCHUNK):
        scores_t = _tc_scores_t(x, wt, ct, c)
        topi_t, topv_t = _sc_topk(scores_t)
        tis.append(topi_t.T)
        tvs.append(topv_t.T)
    return (jnp.concatenate(tis, axis=0), jnp.concatenate(tvs, axis=0))


# speedup vs baseline: 2.6640x; 1.0140x over previous
import functools

import jax
import jax.numpy as jnp
from jax import lax
from jax.experimental import pallas as pl
from jax.experimental.pallas import tpu as pltpu
from jax.experimental.pallas import tpu_sc as plsc

N_EXPERTS = 64
TOPK = 8
BT = 1024
NW = 32
L = 16


def _tc_scores_body(x_ref, wt_ref, st_ref):
    s = jnp.dot(x_ref[...], wt_ref[...], preferred_element_type=jnp.float32)
    e = jnp.exp(s)
    p = e / jnp.sum(e, axis=-1, keepdims=True)
    st_ref[...] = p.T


def _tc_scores_t(x, wt, chunk_tokens, chunk_idx):
    dim = x.shape[1]
    blk0 = chunk_idx * (chunk_tokens // BT)
    return pl.pallas_call(
        _tc_scores_body,
        grid=(chunk_tokens // BT,),
        in_specs=[
            pl.BlockSpec((BT, dim), lambda i: (blk0 + i, 0)),
            pl.BlockSpec((dim, N_EXPERTS), lambda i: (0, 0)),
        ],
        out_specs=pl.BlockSpec((N_EXPERTS, BT), lambda i: (0, i)),
        out_shape=jax.ShapeDtypeStruct((N_EXPERTS, chunk_tokens), jnp.float32),
    )(x, wt)


def _sc_topk(scores_t):
    n_tokens = scores_t.shape[1]
    nt = n_tokens // NW

    @functools.partial(
        pl.kernel,
        mesh=plsc.VectorSubcoreMesh(core_axis_name="c", subcore_axis_name="s"),
        out_type=[
            jax.ShapeDtypeStruct((TOPK, n_tokens), jnp.int32),
            jax.ShapeDtypeStruct((TOPK, n_tokens), jnp.float32),
        ],
        scratch_types=[
            pltpu.VMEM((N_EXPERTS, nt), jnp.float32),
            pltpu.VMEM((TOPK, nt), jnp.int32),
            pltpu.VMEM((TOPK, nt), jnp.float32),
        ],
    )
    def topk_kernel(s_hbm, topi_hbm, topv_hbm, sbuf, ibuf, vbuf):
        wid = lax.axis_index("s") * 2 + lax.axis_index("c")
        base = wid * nt
        pltpu.sync_copy(s_hbm.at[:, pl.ds(base, nt)], sbuf)

        GPI = 1

        def group(g, _):
            lo = g * L
            t = [jnp.full((L,), -1.0, jnp.float32) for _ in range(TOPK)]
            for e in range(N_EXPERTS):
                pv = sbuf[e, pl.ds(lo, L)]
                kb = ((lax.bitcast_convert_type(pv, jnp.int32)
                       & -N_EXPERTS) | (N_EXPERTS - 1 - e))
                c = lax.bitcast_convert_type(kb, jnp.float32)
                for j in range(TOPK):
                    hi = jnp.maximum(t[j], c)
                    c = jnp.minimum(t[j], c)
                    t[j] = hi
            for j in range(TOPK):
                tb = lax.bitcast_convert_type(t[j], jnp.int32)
                ibuf[j, pl.ds(lo, L)] = (
                    (N_EXPERTS - 1) - (tb & (N_EXPERTS - 1)))
                vbuf[j, pl.ds(lo, L)] = lax.bitcast_convert_type(
                    tb & -N_EXPERTS, jnp.float32)
            return 0

        lax.fori_loop(0, nt // L, group, 0)
        pltpu.sync_copy(ibuf, topi_hbm.at[:, pl.ds(base, nt)])
        pltpu.sync_copy(vbuf, topv_hbm.at[:, pl.ds(base, nt)])

    return topk_kernel(scores_t)


NCHUNK = 1


@jax.jit
def kernel(x, W):
    n_tokens = x.shape[0]
    wt = W.T
    ct = n_tokens // NCHUNK
    tis, tvs = [], []
    for c in range(NCHUNK):
        scores_t = _tc_scores_t(x, wt, ct, c)
        topi_t, topv_t = _sc_topk(scores_t)
        tis.append(topi_t.T)
        tvs.append(topv_t.T)
    return (jnp.concatenate(tis, axis=0), jnp.concatenate(tvs, axis=0))


# speedup vs baseline: 2.6840x; 1.0075x over previous
import functools

import jax
import jax.numpy as jnp
from jax import lax
from jax.experimental import pallas as pl
from jax.experimental.pallas import tpu as pltpu
from jax.experimental.pallas import tpu_sc as plsc

N_EXPERTS = 64
TOPK = 8
BT = 1024
NW = 32
L = 16


def _tc_scores_body(x_ref, wt_ref, st_ref):
    s = jnp.dot(x_ref[...], wt_ref[...], preferred_element_type=jnp.float32)
    e = jnp.exp(s)
    p = e / jnp.sum(e, axis=-1, keepdims=True)
    iota = jax.lax.broadcasted_iota(jnp.int32, p.shape, 1)
    kb = ((lax.bitcast_convert_type(p, jnp.int32) & -N_EXPERTS)
          | (N_EXPERTS - 1 - iota))
    st_ref[...] = lax.bitcast_convert_type(kb, jnp.float32).T


def _tc_scores_t(x, wt, chunk_tokens, chunk_idx):
    dim = x.shape[1]
    blk0 = chunk_idx * (chunk_tokens // BT)
    return pl.pallas_call(
        _tc_scores_body,
        grid=(chunk_tokens // BT,),
        in_specs=[
            pl.BlockSpec((BT, dim), lambda i: (blk0 + i, 0)),
            pl.BlockSpec((dim, N_EXPERTS), lambda i: (0, 0)),
        ],
        out_specs=pl.BlockSpec((N_EXPERTS, BT), lambda i: (0, i)),
        out_shape=jax.ShapeDtypeStruct((N_EXPERTS, chunk_tokens), jnp.float32),
    )(x, wt)


def _sc_topk(scores_t):
    n_tokens = scores_t.shape[1]
    nt = n_tokens // NW

    @functools.partial(
        pl.kernel,
        mesh=plsc.VectorSubcoreMesh(core_axis_name="c", subcore_axis_name="s"),
        out_type=[
            jax.ShapeDtypeStruct((TOPK, n_tokens), jnp.int32),
            jax.ShapeDtypeStruct((TOPK, n_tokens), jnp.float32),
        ],
        scratch_types=[
            pltpu.VMEM((N_EXPERTS, nt), jnp.float32),
            pltpu.VMEM((TOPK, nt), jnp.int32),
            pltpu.VMEM((TOPK, nt), jnp.float32),
        ],
    )
    def topk_kernel(s_hbm, topi_hbm, topv_hbm, sbuf, ibuf, vbuf):
        wid = lax.axis_index("s") * 2 + lax.axis_index("c")
        base = wid * nt
        pltpu.sync_copy(s_hbm.at[:, pl.ds(base, nt)], sbuf)

        GPI = 1

        def group(g, _):
            lo = g * L
            t = [jnp.full((L,), -1.0, jnp.float32) for _ in range(TOPK)]
            for e in range(N_EXPERTS):
                c = sbuf[e, pl.ds(lo, L)]
                for j in range(TOPK):
                    hi = jnp.maximum(t[j], c)
                    c = jnp.minimum(t[j], c)
                    t[j] = hi
            for j in range(TOPK):
                tb = lax.bitcast_convert_type(t[j], jnp.int32)
                ibuf[j, pl.ds(lo, L)] = (
                    (N_EXPERTS - 1) - (tb & (N_EXPERTS - 1)))
                vbuf[j, pl.ds(lo, L)] = lax.bitcast_convert_type(
                    tb & -N_EXPERTS, jnp.float32)
            return 0

        lax.fori_loop(0, nt // L, group, 0)
        pltpu.sync_copy(ibuf, topi_hbm.at[:, pl.ds(base, nt)])
        pltpu.sync_copy(vbuf, topv_hbm.at[:, pl.ds(base, nt)])

    return topk_kernel(scores_t)


NCHUNK = 1


@jax.jit
def kernel(x, W):
    n_tokens = x.shape[0]
    wt = W.T
    ct = n_tokens // NCHUNK
    tis, tvs = [], []
    for c in range(NCHUNK):
        scores_t = _tc_scores_t(x, wt, ct, c)
        topi_t, topv_t = _sc_topk(scores_t)
        tis.append(topi_t.T)
        tvs.append(topv_t.T)
    return (jnp.concatenate(tis, axis=0), jnp.concatenate(tvs, axis=0))
